# Initial kernel scaffold; baseline (speedup 1.0000x reference)
import jax
import jax.numpy as jnp
from jax.experimental import pallas as pl


def kernel(x, module_indices, W, b):
    raise NotImplementedError("write your pallas kernel here")


# speedup vs baseline: 3.8305x; 3.8305x over previous
import jax
import jax.numpy as jnp
from jax.experimental import pallas as pl
from jax.experimental.pallas import tpu as pltpu

_E = 64
_D = 1024
_N = 128


def _moe_body(ids_ref, nused_ref, m_ref, x_ref, w_ref, b_ref, o_ref):
    i = pl.program_id(0)

    @pl.when(i == 0)
    def _init():
        o_ref[...] = jnp.zeros_like(o_ref)

    @pl.when(i < nused_ref[0])
    def _step():
        e = ids_ref[i]
        xw = jax.lax.dot_general(
            x_ref[...], w_ref[0],
            dimension_numbers=(((1,), (1,)), ((), ())),
            preferred_element_type=jnp.float32,
        )
        contrib = xw + b_ref[0]
        mask = m_ref[...] == e
        o_ref[...] += jnp.where(mask, contrib, 0.0)


def kernel(x, module_indices, W, b):
    m = module_indices.astype(jnp.int32)
    s = jnp.sort(m)
    is_new = jnp.concatenate([jnp.ones((1,), jnp.bool_), s[1:] != s[:-1]])
    pos = jnp.cumsum(is_new.astype(jnp.int32)) - 1
    ids = jnp.full((_E,), s[-1], jnp.int32).at[pos].set(s)
    n_used = pos[-1:] + 1

    m2d = m.reshape(_N, 1)
    b3 = b.reshape(_E, 1, _D)

    grid_spec = pltpu.PrefetchScalarGridSpec(
        num_scalar_prefetch=2,
        grid=(_E,),
        in_specs=[
            pl.BlockSpec((_N, 1), lambda i, ids, nu: (0, 0)),
            pl.BlockSpec((_N, _D), lambda i, ids, nu: (0, 0)),
            pl.BlockSpec((1, _D, _D), lambda i, ids, nu: (ids[i], 0, 0)),
            pl.BlockSpec((1, 1, _D), lambda i, ids, nu: (ids[i], 0, 0)),
        ],
        out_specs=pl.BlockSpec((_N, _D), lambda i, ids, nu: (0, 0)),
    )

    out = pl.pallas_call(
        _moe_body,
        grid_spec=grid_spec,
        out_shape=jax.ShapeDtypeStruct((_N, _D), jnp.float32),
        compiler_params=pltpu.CompilerParams(
            dimension_semantics=("arbitrary",),
        ),
    )(ids, n_used, m2d, x, W, b3)
    return out
